# baseline (device time: 101599 ns/iter reference)
import jax
import jax.numpy as jnp
from jax import lax
from jax.experimental import pallas as pl
from jax.experimental.pallas import tpu as pltpu

N_DEV = 8
N_PART = 3
PART_SIZES = (176, 168, 168)
PART_OFFS = (0, 176, 344)
DIM_ORDERS = ((0, 1, 2), (1, 2, 0), (2, 0, 1))


def kernel(x, w_mat, scale_x, scale_w):
    m_per, k = x.shape
    _, n_per = w_mat.shape

    s = (scale_x.reshape(-1)[:1] * scale_w.reshape(-1)[:1]).astype(jnp.float32)

    def body(x_ref, w_ref, s_ref, out_ref, *scratch):
        bufs = scratch[0:N_PART]
        sends = scratch[N_PART:2 * N_PART]
        recvs = scratch[2 * N_PART:3 * N_PART]
        w8_ref = scratch[3 * N_PART]

        my = lax.axis_index("i")

        zc = my // 4
        q = lax.rem(my, 4)
        xc = jnp.where((q == 1) | (q == 2), 1, 0)
        yc = jnp.where(q >= 2, 1, 0)
        coords = (xc, yc, zc)

        def make_id(c):
            return 4 * c[2] + 2 * c[1] + jnp.bitwise_xor(c[0], c[1])

        def flipped_id(dims):
            c = list(coords)
            for d in dims:
                c[d] = 1 - c[d]
            return make_id(c)

        neighbors = [flipped_id((d,)) for d in range(3)]

        barrier_sem = pltpu.get_barrier_semaphore()
        for nbr in neighbors:
            pl.semaphore_signal(
                barrier_sem, inc=1,
                device_id=(nbr,), device_id_type=pl.DeviceIdType.MESH,
            )
        pl.semaphore_wait(barrier_sem, 3)

        def rank_origin(p, r):
            dims = tuple(
                DIM_ORDERS[p][i] for i in range(3) if (r >> i) & 1)
            return flipped_id(dims)

        def xfer(p, step):
            lo, hi = 2 ** step, 2 ** (step + 1)
            return pltpu.make_async_remote_copy(
                src_ref=bufs[p].at[0:lo],
                dst_ref=bufs[p].at[lo:hi],
                send_sem=sends[p].at[step],
                recv_sem=recvs[p].at[step],
                device_id=(neighbors[DIM_ORDERS[p][step]],),
                device_id_type=pl.DeviceIdType.MESH,
            )

        def store(p, r):
            acc = lax.dot_general(
                bufs[p][r], w8_ref[...],
                (((1,), (0,)), ((), ())),
                preferred_element_type=jnp.float32,
            )
            g = rank_origin(p, r)
            out_ref[pl.ds(g * m_per + PART_OFFS[p], PART_SIZES[p]), :] = (
                acc * s_ref[0])

        for p in range(N_PART):
            bufs[p][0] = x_ref[
                PART_OFFS[p]:PART_OFFS[p] + PART_SIZES[p], :].astype(
                    jnp.float8_e4m3fn)
        for p in range(N_PART):
            xfer(p, 0).start()
        w8_ref[...] = w_ref[...].astype(jnp.float8_e5m2)
        for p in range(N_PART):
            store(p, 0)

        for step in range(3):
            for p in range(N_PART):
                xfer(p, step).wait_recv()
                if step < 2:
                    xfer(p, step + 1).start()
            for p in range(N_PART):
                for r in range(2 ** step, 2 ** (step + 1)):
                    store(p, r)

        for p in range(N_PART):
            for step in range(3):
                xfer(p, step).wait_send()

    sems = pltpu.SemaphoreType.DMA((3,))
    return pl.pallas_call(
        body,
        out_shape=jax.ShapeDtypeStruct((N_DEV * m_per, n_per), jnp.float32),
        in_specs=[
            pl.BlockSpec(memory_space=pltpu.VMEM),
            pl.BlockSpec(memory_space=pltpu.VMEM),
            pl.BlockSpec(memory_space=pltpu.SMEM),
        ],
        out_specs=pl.BlockSpec(memory_space=pltpu.VMEM),
        scratch_shapes=[
            pltpu.VMEM((N_DEV, PART_SIZES[p], k), jnp.float8_e4m3fn)
            for p in range(N_PART)
        ] + [sems] * (2 * N_PART) + [
            pltpu.VMEM((k, n_per), jnp.float8_e5m2),
        ],
        compiler_params=pltpu.CompilerParams(
            collective_id=0, vmem_limit_bytes=100 * 1024 * 1024),
    )(x, w_mat, s)


# device time: 90055 ns/iter; 1.1282x vs baseline; 1.1282x over previous
import jax
import jax.numpy as jnp
from jax import lax
from jax.experimental import pallas as pl
from jax.experimental.pallas import tpu as pltpu

N_DEV = 8
N_PART = 3
PART_SIZES = (176, 168, 168)
PART_OFFS = (0, 176, 344)
DIM_ORDERS = ((0, 1, 2), (1, 2, 0), (2, 0, 1))


def kernel(x, w_mat, scale_x, scale_w):
    m_per, k = x.shape
    _, n_per = w_mat.shape

    s = (scale_x.reshape(-1)[:1] * scale_w.reshape(-1)[:1]).astype(jnp.float32)

    def body(x_ref, w_ref, s_ref, out_ref, *scratch):
        bufs = scratch[0:N_PART]
        sends = scratch[N_PART:2 * N_PART]
        recvs = scratch[2 * N_PART:3 * N_PART]
        w8_ref = scratch[3 * N_PART]

        my = lax.axis_index("i")

        zc = my // 4
        q = lax.rem(my, 4)
        xc = jnp.where((q == 1) | (q == 2), 1, 0)
        yc = jnp.where(q >= 2, 1, 0)
        coords = (xc, yc, zc)

        def make_id(c):
            return 4 * c[2] + 2 * c[1] + jnp.bitwise_xor(c[0], c[1])

        def flipped_id(dims):
            c = list(coords)
            for d in dims:
                c[d] = 1 - c[d]
            return make_id(c)

        neighbors = [flipped_id((d,)) for d in range(3)]

        barrier_sem = pltpu.get_barrier_semaphore()
        for nbr in neighbors:
            pl.semaphore_signal(
                barrier_sem, inc=1,
                device_id=(nbr,), device_id_type=pl.DeviceIdType.MESH,
            )
        pl.semaphore_wait(barrier_sem, 3)

        def subsets(p, step):
            used = DIM_ORDERS[p][:step]
            out = [()]
            for d in used:
                out = out + [m + (d,) for m in out]
            return out

        def jidx(p, mask):
            return sum(
                2 ** i for i, d in enumerate(DIM_ORDERS[p]) if d in mask)

        def xfer(p, step, j, g):
            d = DIM_ORDERS[p][step]
            return pltpu.make_async_remote_copy(
                src_ref=bufs[p].at[g],
                dst_ref=bufs[p].at[g],
                send_sem=sends[p].at[step, j],
                recv_sem=recvs[p].at[step, j],
                device_id=(neighbors[d],),
                device_id_type=pl.DeviceIdType.MESH,
            )

        def store(p, g):
            acc = lax.dot_general(
                bufs[p][g], w8_ref[...],
                (((1,), (0,)), ((), ())),
                preferred_element_type=jnp.float32,
            )
            out_ref[pl.ds(g * m_per + PART_OFFS[p], PART_SIZES[p]), :] = (
                acc * s_ref[0])

        for p in range(N_PART):
            bufs[p][my] = x_ref[
                PART_OFFS[p]:PART_OFFS[p] + PART_SIZES[p], :].astype(
                    jnp.float8_e4m3fn)
        for step in range(3):
            for p in range(N_PART):
                xfer(p, step, 0, my).start()
        w8_ref[...] = w_ref[...].astype(jnp.float8_e5m2)
        for p in range(N_PART):
            store(p, my)

        for step in range(3):
            masks = {p: subsets(p, step) for p in range(N_PART)}
            for jpos in range(2 ** step):
                for p in range(N_PART):
                    d = DIM_ORDERS[p][step]
                    newmask = masks[p][jpos] + (d,)
                    g = flipped_id(newmask)
                    xfer(p, step, jpos, g).wait_recv()
                    for step2 in range(step + 1, 3):
                        xfer(p, step2, jidx(p, newmask), g).start()
                    store(p, g)

        for p in range(N_PART):
            for step in range(3):
                for j in range(2 ** step):
                    xfer(p, step, j, my).wait_send()

    sems = pltpu.SemaphoreType.DMA((3, 4))
    return pl.pallas_call(
        body,
        out_shape=jax.ShapeDtypeStruct((N_DEV * m_per, n_per), jnp.float32),
        in_specs=[
            pl.BlockSpec(memory_space=pltpu.VMEM),
            pl.BlockSpec(memory_space=pltpu.VMEM),
            pl.BlockSpec(memory_space=pltpu.SMEM),
        ],
        out_specs=pl.BlockSpec(memory_space=pltpu.VMEM),
        scratch_shapes=[
            pltpu.VMEM((N_DEV, PART_SIZES[p], k), jnp.float8_e4m3fn)
            for p in range(N_PART)
        ] + [sems] * (2 * N_PART) + [
            pltpu.VMEM((k, n_per), jnp.float8_e5m2),
        ],
        compiler_params=pltpu.CompilerParams(
            collective_id=0, vmem_limit_bytes=100 * 1024 * 1024),
    )(x, w_mat, s)
